# baseline (speedup 1.0000x reference)
import functools

import jax
import jax.numpy as jnp
from jax import lax
from jax.experimental import pallas as pl
from jax.experimental.pallas import tpu as pltpu
from jax.experimental.pallas import tpu_sc as plsc

N_NODES = 100000
D_FEAT = 128
N_READOUT = 16384

NC = 2
NS = 16
L = 16

HALF = N_READOUT // NC
E_PER_W = N_READOUT // NS
CHUNK = 128
STAGE = E_PER_W + CHUNK
NCHUNK = STAGE // CHUNK
TRASH = HALF
ACC_ROWS = HALF + 8
OUT_PER_W = HALF // NS

_mesh = plsc.VectorSubcoreMesh(
    core_axis_name="c", subcore_axis_name="s", num_cores=NC, num_subcores=NS
)


@functools.partial(
    pl.kernel,
    out_type=jax.ShapeDtypeStruct((N_READOUT, D_FEAT), jnp.float32),
    mesh=_mesh,
    scratch_types=[
        pltpu.VMEM((STAGE,), jnp.int32),
        pltpu.VMEM((STAGE,), jnp.int32),
        pltpu.VMEM((CHUNK,), jnp.int32),
        pltpu.VMEM((3, CHUNK, D_FEAT), jnp.float32),
        pltpu.VMEM_SHARED((ACC_ROWS, D_FEAT), jnp.float32),
        pltpu.SemaphoreType.DMA,
        pltpu.SemaphoreType.DMA,
        pltpu.SemaphoreType.DMA,
    ],
)
def _readout_kernel(x_hbm, src_hbm, dst_hbm, out_hbm,
                    src_v, dst_v, dloc_v, rows_v,
                    acc_sh, sem_a, sem_b, sem_c):
    c = lax.axis_index("c")
    s = lax.axis_index("s")
    sems = (sem_a, sem_b, sem_c)

    one = jnp.ones((L,), jnp.int32)
    zint = jnp.zeros((L,), jnp.int32)

    def _section(t, cnt):
        off = pl.multiple_of(t * E_PER_W, 8)
        pltpu.sync_copy(dst_hbm.at[pl.ds(off, E_PER_W)],
                        dst_v.at[pl.ds(0, E_PER_W)])
        for i in range(E_PER_W // L):
            d = dst_v[pl.ds(i * L, L)]
            cnt = cnt + jnp.where(d < HALF, one, zint)
        return cnt

    count = lax.fori_loop(0, NS, _section, jnp.zeros((L,), jnp.int32))

    zero = jnp.zeros((L,), jnp.float32)

    def _zrow(i, carry):
        for q in range(D_FEAT // L):
            rows_v[2, i, pl.ds(q * L, L)] = zero
        return carry

    lax.fori_loop(0, CHUNK, _zrow, 0)
    for r in range(OUT_PER_W // CHUNK):
        pltpu.sync_copy(rows_v.at[2],
                        acc_sh.at[pl.ds(s * OUT_PER_W + r * CHUNK, CHUNK)])
    plsc.subcore_barrier()

    cnt0 = count[0]
    cut_hi = jnp.minimum(16 * cnt0 + 16, N_READOUT)
    cut_lo = jnp.maximum(16 * cnt0 - 16, 0)

    n_c = cut_hi + c * (N_READOUT - cut_lo - cut_hi)
    lo_c = c * cut_lo
    my_lo = lo_c + (s * n_c) // NS
    my_hi = lo_c + ((s + 1) * n_c) // NS
    base = jnp.minimum((my_lo // 8) * 8, N_READOUT - STAGE)
    base = pl.multiple_of(base, 8)

    pltpu.sync_copy(src_hbm.at[pl.ds(base, STAGE)], src_v)
    pltpu.sync_copy(dst_hbm.at[pl.ds(base, STAGE)], dst_v)

    def _active(j):
        return base + j * CHUNK < my_hi

    def _gather(j):
        return pltpu.async_copy(
            x_hbm.at[src_v.at[pl.ds(j * CHUNK, CHUNK)]],
            rows_v.at[j % 3], sems[j % 3],
        )

    def _wait(j):
        pltpu.make_async_copy(
            x_hbm.at[src_v.at[pl.ds(j * CHUNK, CHUNK)]],
            rows_v.at[j % 3], sems[j % 3],
        ).wait()

    @pl.when(_active(0))
    def _():
        _gather(0)

    @pl.when(_active(1))
    def _():
        _gather(1)

    base_local = c * HALF
    lane = lax.iota(jnp.int32, L)
    for j in range(NCHUNK):
        if j + 2 < NCHUNK:
            @pl.when(_active(j + 2))
            def _():
                _gather(j + 2)

        @pl.when(_active(j))
        def _():
            for i in range(CHUNK // L):
                pos = base + j * CHUNK + i * L + lane
                d = dst_v[pl.ds(j * CHUNK + i * L, L)]
                dl = d - base_local
                ok = (pos >= my_lo) & (pos < my_hi) & (dl >= 0) & (dl < HALF)
                dloc_v[pl.ds(i * L, L)] = jnp.where(ok, dl, TRASH)
            _wait(j)
            pltpu.sync_copy(rows_v.at[j % 3], acc_sh.at[dloc_v], add=True)

    plsc.subcore_barrier()

    out_base = c * HALF + s * OUT_PER_W
    pltpu.sync_copy(
        acc_sh.at[pl.ds(s * OUT_PER_W, OUT_PER_W)],
        out_hbm.at[pl.ds(out_base, OUT_PER_W)],
    )


def kernel(x, readout_src, readout_dst):
    return _readout_kernel(
        x, readout_src.astype(jnp.int32), readout_dst.astype(jnp.int32)
    )


# speedup vs baseline: 1.2383x; 1.2383x over previous
import functools

import jax
import jax.numpy as jnp
from jax import lax
from jax.experimental import pallas as pl
from jax.experimental.pallas import tpu as pltpu
from jax.experimental.pallas import tpu_sc as plsc

N_NODES = 100000
D_FEAT = 128
N_READOUT = 16384

NC = 2
NS = 16
L = 16

HALF = N_READOUT // NC
E_PER_W = N_READOUT // NS
CHUNK = 128
STAGE = E_PER_W + CHUNK
NCHUNK = STAGE // CHUNK
TRASH = HALF
ACC_ROWS = HALF + 8
OUT_PER_W = HALF // NS

_mesh = plsc.VectorSubcoreMesh(
    core_axis_name="c", subcore_axis_name="s", num_cores=NC, num_subcores=NS
)


@functools.partial(
    pl.kernel,
    out_type=jax.ShapeDtypeStruct((N_READOUT, D_FEAT), jnp.float32),
    mesh=_mesh,
    scratch_types=[
        pltpu.VMEM((STAGE,), jnp.int32),
        pltpu.VMEM((STAGE,), jnp.int32),
        pltpu.VMEM((CHUNK,), jnp.int32),
        pltpu.VMEM((3, CHUNK, D_FEAT), jnp.float32),
        pltpu.VMEM_SHARED((ACC_ROWS, D_FEAT), jnp.float32),
        pltpu.SemaphoreType.DMA,
        pltpu.SemaphoreType.DMA,
        pltpu.SemaphoreType.DMA,
    ],
)
def _readout_kernel(x_hbm, src_hbm, dst_hbm, samp_hbm, out_hbm,
                    src_v, dst_v, dloc_v, rows_v,
                    acc_sh, sem_a, sem_b, sem_c):
    c = lax.axis_index("c")
    s = lax.axis_index("s")
    sems = (sem_a, sem_b, sem_c)

    pltpu.sync_copy(samp_hbm, dst_v.at[pl.ds(0, N_READOUT // 16)])
    one = jnp.ones((L,), jnp.int32)
    zint = jnp.zeros((L,), jnp.int32)
    count = jnp.zeros((L,), jnp.int32)
    for i in range(N_READOUT // 16 // L):
        d = dst_v[pl.ds(i * L, L)]
        count = count + jnp.where(d < HALF, one, zint)

    zero = jnp.zeros((L,), jnp.float32)

    def _zrow(i, carry):
        for q in range(D_FEAT // L):
            rows_v[2, i, pl.ds(q * L, L)] = zero
        return carry

    lax.fori_loop(0, CHUNK, _zrow, 0)
    for r in range(OUT_PER_W // CHUNK):
        pltpu.sync_copy(rows_v.at[2],
                        acc_sh.at[pl.ds(s * OUT_PER_W + r * CHUNK, CHUNK)])
    plsc.subcore_barrier()

    cnt0 = count[0]
    cut_hi = jnp.minimum(256 * cnt0 + 256, N_READOUT)
    cut_lo = jnp.maximum(256 * cnt0 - 256, 0)

    n_c = cut_hi + c * (N_READOUT - cut_lo - cut_hi)
    lo_c = c * cut_lo
    my_lo = lo_c + (s * n_c) // NS
    my_hi = lo_c + ((s + 1) * n_c) // NS
    base = jnp.minimum((my_lo // 8) * 8, N_READOUT - STAGE)
    base = pl.multiple_of(base, 8)

    pltpu.sync_copy(src_hbm.at[pl.ds(base, STAGE)], src_v)
    pltpu.sync_copy(dst_hbm.at[pl.ds(base, STAGE)], dst_v)

    def _active(j):
        return base + j * CHUNK < my_hi

    def _gather(j):
        return pltpu.async_copy(
            x_hbm.at[src_v.at[pl.ds(j * CHUNK, CHUNK)]],
            rows_v.at[j % 3], sems[j % 3],
        )

    def _wait(j):
        pltpu.make_async_copy(
            x_hbm.at[src_v.at[pl.ds(j * CHUNK, CHUNK)]],
            rows_v.at[j % 3], sems[j % 3],
        ).wait()

    @pl.when(_active(0))
    def _():
        _gather(0)

    @pl.when(_active(1))
    def _():
        _gather(1)

    base_local = c * HALF
    lane = lax.iota(jnp.int32, L)
    for j in range(NCHUNK):
        if j + 2 < NCHUNK:
            @pl.when(_active(j + 2))
            def _():
                _gather(j + 2)

        @pl.when(_active(j))
        def _():
            for i in range(CHUNK // L):
                pos = base + j * CHUNK + i * L + lane
                d = dst_v[pl.ds(j * CHUNK + i * L, L)]
                dl = d - base_local
                ok = (pos >= my_lo) & (pos < my_hi) & (dl >= 0) & (dl < HALF)
                dloc_v[pl.ds(i * L, L)] = jnp.where(ok, dl, TRASH)
            _wait(j)
            pltpu.sync_copy(rows_v.at[j % 3], acc_sh.at[dloc_v], add=True)

    plsc.subcore_barrier()

    out_base = c * HALF + s * OUT_PER_W
    pltpu.sync_copy(
        acc_sh.at[pl.ds(s * OUT_PER_W, OUT_PER_W)],
        out_hbm.at[pl.ds(out_base, OUT_PER_W)],
    )


def kernel(x, readout_src, readout_dst):
    dst32 = readout_dst.astype(jnp.int32)
    return _readout_kernel(x, readout_src.astype(jnp.int32), dst32,
                           dst32[::16])


# speedup vs baseline: 1.3462x; 1.0871x over previous
import functools

import jax
import jax.numpy as jnp
from jax import lax
from jax.experimental import pallas as pl
from jax.experimental.pallas import tpu as pltpu
from jax.experimental.pallas import tpu_sc as plsc

N_NODES = 100000
D_FEAT = 128
N_READOUT = 16384

NC = 2
NS = 16
L = 16

HALF = N_READOUT // NC
E_PER_W = N_READOUT // NS
CHUNK = 128
STAGE = E_PER_W + CHUNK
NCHUNK = STAGE // CHUNK
NSAMP = N_READOUT // 16
TRASH = HALF
ACC_ROWS = HALF + 8
OUT_PER_W = HALF // NS

_mesh = plsc.VectorSubcoreMesh(
    core_axis_name="c", subcore_axis_name="s", num_cores=NC, num_subcores=NS
)


@functools.partial(
    pl.kernel,
    out_type=jax.ShapeDtypeStruct((N_READOUT, D_FEAT), jnp.float32),
    mesh=_mesh,
    scratch_types=[
        pltpu.VMEM((STAGE,), jnp.int32),
        pltpu.VMEM((STAGE,), jnp.int32),
        pltpu.VMEM((CHUNK,), jnp.int32),
        pltpu.VMEM((CHUNK,), jnp.int32),
        pltpu.VMEM((CHUNK,), jnp.int32),
        pltpu.VMEM((3, CHUNK, D_FEAT), jnp.float32),
        pltpu.VMEM_SHARED((ACC_ROWS, D_FEAT), jnp.float32),
        pltpu.SemaphoreType.DMA,
        pltpu.SemaphoreType.DMA,
        pltpu.SemaphoreType.DMA,
        pltpu.SemaphoreType.DMA,
        pltpu.SemaphoreType.DMA,
        pltpu.SemaphoreType.DMA,
        pltpu.SemaphoreType.DMA,
    ],
)
def _readout_kernel(x_hbm, src_hbm, dst_hbm, samp_hbm, out_hbm,
                    src_v, dst_v, dloc_a, dloc_b, dloc_c, rows_v, acc_sh,
                    gsem_a, gsem_b, gsem_c, wsem_a, wsem_b, wsem_c, zsem):
    c = lax.axis_index("c")
    s = lax.axis_index("s")
    gsems = (gsem_a, gsem_b, gsem_c)
    wsems = (wsem_a, wsem_b, wsem_c)
    dlocs = (dloc_a, dloc_b, dloc_c)

    pltpu.sync_copy(samp_hbm, dst_v.at[pl.ds(0, NSAMP)])
    one = jnp.ones((L,), jnp.int32)
    zint = jnp.zeros((L,), jnp.int32)
    count = jnp.zeros((L,), jnp.int32)
    for i in range(NSAMP // L):
        d = dst_v[pl.ds(i * L, L)]
        count = count + jnp.where(d < HALF, one, zint)

    zero = jnp.zeros((L,), jnp.float32)

    def _zrow(i, carry):
        for q in range(D_FEAT // L):
            rows_v[2, i, pl.ds(q * L, L)] = zero
        return carry

    lax.fori_loop(0, CHUNK, _zrow, 0)
    for r in range(OUT_PER_W // CHUNK):
        pltpu.async_copy(rows_v.at[2],
                         acc_sh.at[pl.ds(s * OUT_PER_W + r * CHUNK, CHUNK)],
                         zsem)

    cnt0 = count[0]
    cut_hi = jnp.minimum(256 * cnt0 + 256, N_READOUT)
    cut_lo = jnp.maximum(256 * cnt0 - 256, 0)

    n_c = cut_hi + c * (N_READOUT - cut_lo - cut_hi)
    lo_c = c * cut_lo
    my_lo = lo_c + (s * n_c) // NS
    my_hi = lo_c + ((s + 1) * n_c) // NS
    base = jnp.minimum((my_lo // 8) * 8, N_READOUT - STAGE)
    base = pl.multiple_of(base, 8)

    pltpu.sync_copy(src_hbm.at[pl.ds(base, STAGE)], src_v)
    pltpu.sync_copy(dst_hbm.at[pl.ds(base, STAGE)], dst_v)

    def _active(j):
        return base + j * CHUNK < my_hi

    def _gather(j):
        pltpu.async_copy(
            x_hbm.at[src_v.at[pl.ds(j * CHUNK, CHUNK)]],
            rows_v.at[j % 3], gsems[j % 3],
        )

    def _gwait(j):
        pltpu.make_async_copy(
            x_hbm.at[src_v.at[pl.ds(j * CHUNK, CHUNK)]],
            rows_v.at[j % 3], gsems[j % 3],
        ).wait()

    def _scatter(j):
        pltpu.async_copy(rows_v.at[j % 3], acc_sh.at[dlocs[j % 3]],
                         wsems[j % 3], add=True)

    def _swait(j):
        pltpu.make_async_copy(rows_v.at[j % 3], acc_sh.at[dlocs[j % 3]],
                              wsems[j % 3]).wait()

    @pl.when(_active(0))
    def _():
        _gather(0)

    @pl.when(_active(1))
    def _():
        _gather(1)

    for r in range(OUT_PER_W // CHUNK):
        pltpu.make_async_copy(
            rows_v.at[2],
            acc_sh.at[pl.ds(s * OUT_PER_W + r * CHUNK, CHUNK)], zsem
        ).wait()
    plsc.subcore_barrier()

    base_local = c * HALF
    lane = lax.iota(jnp.int32, L)
    for j in range(NCHUNK):
        if j + 2 < NCHUNK and j >= 1:
            @pl.when(_active(j - 1))
            def _():
                _swait(j - 1)
        if j + 2 < NCHUNK:
            @pl.when(_active(j + 2))
            def _():
                _gather(j + 2)

        @pl.when(_active(j))
        def _():
            dloc = dlocs[j % 3]
            for i in range(CHUNK // L):
                pos = base + j * CHUNK + i * L + lane
                d = dst_v[pl.ds(j * CHUNK + i * L, L)]
                dl = d - base_local
                ok = (pos >= my_lo) & (pos < my_hi) & (dl >= 0) & (dl < HALF)
                dloc[pl.ds(i * L, L)] = jnp.where(ok, dl, TRASH)
            _gwait(j)
            _scatter(j)

    for j in range(max(NCHUNK - 3, 0), NCHUNK):
        @pl.when(_active(j))
        def _():
            _swait(j)

    plsc.subcore_barrier()

    out_base = c * HALF + s * OUT_PER_W
    pltpu.sync_copy(
        acc_sh.at[pl.ds(s * OUT_PER_W, OUT_PER_W)],
        out_hbm.at[pl.ds(out_base, OUT_PER_W)],
    )


def kernel(x, readout_src, readout_dst):
    dst32 = readout_dst.astype(jnp.int32)
    return _readout_kernel(x, readout_src.astype(jnp.int32), dst32,
                           dst32[::16])
